# baseline (speedup 1.0000x reference)
import functools

import jax
import jax.numpy as jnp
from jax import lax
from jax.experimental import pallas as pl
from jax.experimental.pallas import tpu as pltpu
from jax.experimental.pallas import tpu_sc as plsc

_B = 4096
_L = 200
_D = 128
_FC = 2048
_NW = 32
_RPW = _B // _NW
_NREG = _D // 16
_G = 8


def _pool_sc(h_tok, h_len, p_tok, p_len, table):
    mesh = plsc.VectorSubcoreMesh(core_axis_name="c", subcore_axis_name="s")

    @functools.partial(
        pl.kernel,
        mesh=mesh,
        out_type=(
            jax.ShapeDtypeStruct((_B, _D), jnp.float32),
            jax.ShapeDtypeStruct((_B, _D), jnp.float32),
        ),
        scratch_types=[
            pltpu.VMEM((_RPW * _L,), jnp.int32),
            pltpu.VMEM((_RPW + 16,), jnp.int32),
            pltpu.VMEM((_L, _D), jnp.float32),
            pltpu.VMEM((_L, _D), jnp.float32),
            pltpu.VMEM((_RPW, _D), jnp.float32),
            pltpu.SemaphoreType.DMA,
            pltpu.SemaphoreType.DMA,
        ],
    )
    def k(h_tok_hbm, h_len_hbm, p_tok_hbm, p_len_hbm, table_hbm,
          h_out_hbm, p_out_hbm, idx_v, len_v, buf0, buf1, out_v, sem0, sem1):
        wid = lax.axis_index("s") * 2 + lax.axis_index("c")
        base = wid * _RPW

        def do_side(tok_hbm, lens_hbm, out_hbm):
            pltpu.sync_copy(tok_hbm.at[pl.ds(base * _L, _RPW * _L)], idx_v)
            pltpu.sync_copy(lens_hbm.at[pl.ds(base, _RPW)],
                            len_v.at[pl.ds(0, _RPW)])

            def nchunks(r):
                ln = len_v[pl.ds(r, 16)][0]
                return ln >> 4, lax.shift_right_logical(ln + 7, 3) << 3

            def gather(r, nc, buf, sem):
                nc16, tot = nc

                def cbody(c, _):
                    e = _L * r + 2 * _G * c
                    pltpu.async_copy(
                        table_hbm.at[idx_v.at[pl.ds(e, 2 * _G)]],
                        buf.at[pl.ds(c * 2 * _G, 2 * _G)], sem)
                    return 0
                lax.fori_loop(0, nc16, cbody, 0)

                @pl.when(tot > (nc16 << 4))
                def _():
                    e = pl.multiple_of(_L * r + (nc16 << 4), _G)
                    pltpu.async_copy(
                        table_hbm.at[idx_v.at[pl.ds(e, _G)]],
                        buf.at[pl.ds(pl.multiple_of(nc16 << 4, _G), _G)], sem)

                @pl.when(tot > (nc16 << 4) + _G)
                def _():
                    e = pl.multiple_of(_L * r + (nc16 << 4) + _G, _G)
                    pltpu.async_copy(
                        table_hbm.at[idx_v.at[pl.ds(e, _G)]],
                        buf.at[pl.ds(pl.multiple_of((nc16 << 4) + _G, _G), _G)],
                        sem)

            def wait(nc, buf, sem):
                nc16, tot = nc

                def cbody(c, _):
                    pltpu.make_async_copy(
                        table_hbm.at[idx_v.at[pl.ds(0, 2 * _G)]],
                        buf.at[pl.ds(0, 2 * _G)], sem).wait()
                    return 0
                lax.fori_loop(0, nc16, cbody, 0)

                @pl.when(tot > (nc16 << 4))
                def _():
                    pltpu.make_async_copy(
                        table_hbm.at[idx_v.at[pl.ds(0, _G)]],
                        buf.at[pl.ds(0, _G)], sem).wait()

                @pl.when(tot > (nc16 << 4) + _G)
                def _():
                    pltpu.make_async_copy(
                        table_hbm.at[idx_v.at[pl.ds(0, _G)]],
                        buf.at[pl.ds(0, _G)], sem).wait()

            def accum(r, buf):
                ln = len_v[pl.ds(r, 16)][0]

                def add_row(j, acc):
                    return tuple(acc[q] + buf[j, pl.ds(16 * q, 16)]
                                 for q in range(_NREG))

                def jbody(j2, acc):
                    return add_row(2 * j2 + 1, add_row(2 * j2, acc))

                acc = lax.fori_loop(
                    0, ln >> 1, jbody,
                    tuple(jnp.zeros((16,), jnp.float32) for _ in range(_NREG)))
                tail = jnp.broadcast_to((ln & 1).astype(jnp.float32), (16,))
                jt = (ln - 1) & ~1
                acc = tuple(acc[q] + buf[jt, pl.ds(16 * q, 16)] * tail
                            for q in range(_NREG))
                inv = 1.0 / jnp.broadcast_to(ln.astype(jnp.float32), (16,))
                for q in range(_NREG):
                    out_v[r, pl.ds(16 * q, 16)] = acc[q] * inv

            gather(0, nchunks(0), buf0, sem0)
            gather(1, nchunks(1), buf1, sem1)

            def rbody(r2, _):
                r0 = 2 * r2
                wait(nchunks(r0), buf0, sem0)
                accum(r0, buf0)

                @pl.when(r0 + 2 < _RPW)
                def _():
                    gather(r0 + 2, nchunks(r0 + 2), buf0, sem0)

                wait(nchunks(r0 + 1), buf1, sem1)
                accum(r0 + 1, buf1)

                @pl.when(r0 + 3 < _RPW)
                def _():
                    gather(r0 + 3, nchunks(r0 + 3), buf1, sem1)

                return 0

            lax.fori_loop(0, _RPW // 2, rbody, 0)
            pltpu.sync_copy(out_v, out_hbm.at[pl.ds(base, _RPW)])

        do_side(h_tok_hbm, h_len_hbm, h_out_hbm)
        do_side(p_tok_hbm, p_len_hbm, p_out_hbm)

    return k(h_tok, h_len, p_tok, p_len, table)


def _mlp_body(hp_ref, hh_ref, w1_ref, b1_ref, w2_ref, b2_ref, w3_ref, b3_ref,
              out_ref):
    hp = hp_ref[...]
    hh = hh_ref[...]
    f = jnp.concatenate([hp, hh, jnp.abs(hp - hh), hp * hh], axis=1)
    x1 = jnp.dot(f, w1_ref[...], preferred_element_type=jnp.float32) + b1_ref[...]
    x2 = jnp.dot(x1, w2_ref[...], preferred_element_type=jnp.float32) + b2_ref[...]
    out_ref[...] = (jnp.dot(x2, w3_ref[...], preferred_element_type=jnp.float32)
                    + b3_ref[...])


def _mlp_tc(hp, hh, W1, b1, W2, b2, W3p, b3p):
    BM = 512
    return pl.pallas_call(
        _mlp_body,
        grid=(_B // BM,),
        in_specs=[
            pl.BlockSpec((BM, _D), lambda i: (i, 0)),
            pl.BlockSpec((BM, _D), lambda i: (i, 0)),
            pl.BlockSpec((4 * _D, _FC), lambda i: (0, 0)),
            pl.BlockSpec((1, _FC), lambda i: (0, 0)),
            pl.BlockSpec((_FC, _FC), lambda i: (0, 0)),
            pl.BlockSpec((1, _FC), lambda i: (0, 0)),
            pl.BlockSpec((_FC, _D), lambda i: (0, 0)),
            pl.BlockSpec((1, _D), lambda i: (0, 0)),
        ],
        out_specs=pl.BlockSpec((BM, _D), lambda i: (i, 0)),
        out_shape=jax.ShapeDtypeStruct((_B, _D), jnp.float32),
    )(hp, hh, W1, b1.reshape(1, _FC), W2, b2.reshape(1, _FC), W3p, b3p)


def kernel(hypothesis_tokens, hypothesis_len, premise_tokens, premise_len,
           emb_table, W1, b1, W2, b2, W3, b3):
    h_tok = hypothesis_tokens.astype(jnp.int32).reshape(_B * _L)
    p_tok = premise_tokens.astype(jnp.int32).reshape(_B * _L)
    h_len = hypothesis_len.astype(jnp.int32)
    p_len = premise_len.astype(jnp.int32)
    hh, hp = _pool_sc(h_tok, h_len, p_tok, p_len, emb_table)
    W3p = jnp.pad(W3, ((0, 0), (0, _D - W3.shape[1])))
    b3p = jnp.pad(b3, (0, _D - b3.shape[0])).reshape(1, _D)
    out = _mlp_tc(hp, hh, W1, b1, W2, b2, W3p, b3p)
    return out[:, :W3.shape[1]]


# speedup vs baseline: 1.0096x; 1.0096x over previous
import functools

import jax
import jax.numpy as jnp
from jax import lax
from jax.experimental import pallas as pl
from jax.experimental.pallas import tpu as pltpu
from jax.experimental.pallas import tpu_sc as plsc

_B = 4096
_L = 200
_D = 128
_FC = 2048
_NW = 32
_RPW = _B // _NW
_NREG = _D // 16
_G = 8


def _pool_sc(h_tok, h_len, p_tok, p_len, table):
    mesh = plsc.VectorSubcoreMesh(core_axis_name="c", subcore_axis_name="s")

    @functools.partial(
        pl.kernel,
        mesh=mesh,
        out_type=(
            jax.ShapeDtypeStruct((_B, _D), jnp.float32),
            jax.ShapeDtypeStruct((_B, _D), jnp.float32),
        ),
        scratch_types=[
            pltpu.VMEM((_RPW * _L,), jnp.int32),
            pltpu.VMEM((_RPW + 16,), jnp.int32),
            pltpu.VMEM((_L, _D), jnp.float32),
            pltpu.VMEM((_L, _D), jnp.float32),
            pltpu.VMEM((_RPW, _D), jnp.float32),
            pltpu.SemaphoreType.DMA,
            pltpu.SemaphoreType.DMA,
        ],
    )
    def k(h_tok_hbm, h_len_hbm, p_tok_hbm, p_len_hbm, table_hbm,
          h_out_hbm, p_out_hbm, idx_v, len_v, buf0, buf1, out_v, sem0, sem1):
        wid = lax.axis_index("s") * 2 + lax.axis_index("c")
        base = wid * _RPW

        def do_side(tok_hbm, lens_hbm, out_hbm):
            pltpu.sync_copy(tok_hbm.at[pl.ds(base * _L, _RPW * _L)], idx_v)
            pltpu.sync_copy(lens_hbm.at[pl.ds(base, _RPW)],
                            len_v.at[pl.ds(0, _RPW)])

            def nchunks(r):
                ln = len_v[pl.ds(r, 16)][0]
                return lax.shift_right_logical(ln + (_G - 1), 3)

            def gather(r, nc, buf, sem):
                def cbody(c, _):
                    e = _L * r + _G * c
                    pltpu.async_copy(
                        table_hbm.at[idx_v.at[pl.ds(e, _G)]],
                        buf.at[pl.ds(c * _G, _G)], sem)
                    return 0
                lax.fori_loop(0, nc, cbody, 0)

            def wait(nc, buf, sem):
                def cbody(c, _):
                    pltpu.make_async_copy(
                        table_hbm.at[idx_v.at[pl.ds(0, _G)]],
                        buf.at[pl.ds(0, _G)], sem).wait()
                    return 0
                lax.fori_loop(0, nc, cbody, 0)

            def accum(r, buf):
                ln = len_v[pl.ds(r, 16)][0]

                def add_row(j, acc):
                    return tuple(acc[q] + buf[j, pl.ds(16 * q, 16)]
                                 for q in range(_NREG))

                def jbody(j2, acc):
                    return add_row(2 * j2 + 1, add_row(2 * j2, acc))

                acc = lax.fori_loop(
                    0, ln >> 1, jbody,
                    tuple(jnp.zeros((16,), jnp.float32) for _ in range(_NREG)))
                tail = jnp.broadcast_to((ln & 1).astype(jnp.float32), (16,))
                jt = (ln - 1) & ~1
                acc = tuple(acc[q] + buf[jt, pl.ds(16 * q, 16)] * tail
                            for q in range(_NREG))
                inv = 1.0 / jnp.broadcast_to(ln.astype(jnp.float32), (16,))
                for q in range(_NREG):
                    out_v[r, pl.ds(16 * q, 16)] = acc[q] * inv

            gather(0, nchunks(0), buf0, sem0)
            gather(1, nchunks(1), buf1, sem1)

            def rbody(r2, _):
                r0 = 2 * r2
                wait(nchunks(r0), buf0, sem0)
                accum(r0, buf0)

                @pl.when(r0 + 2 < _RPW)
                def _():
                    gather(r0 + 2, nchunks(r0 + 2), buf0, sem0)

                wait(nchunks(r0 + 1), buf1, sem1)
                accum(r0 + 1, buf1)

                @pl.when(r0 + 3 < _RPW)
                def _():
                    gather(r0 + 3, nchunks(r0 + 3), buf1, sem1)

                return 0

            lax.fori_loop(0, _RPW // 2, rbody, 0)
            pltpu.sync_copy(out_v, out_hbm.at[pl.ds(base, _RPW)])

        do_side(h_tok_hbm, h_len_hbm, h_out_hbm)
        do_side(p_tok_hbm, p_len_hbm, p_out_hbm)

    return k(h_tok, h_len, p_tok, p_len, table)


def _mlp_body(hp_ref, hh_ref, w1_ref, b1_ref, w2_ref, b2_ref, w3_ref, b3_ref,
              out_ref):
    hp = hp_ref[...]
    hh = hh_ref[...]
    f = jnp.concatenate([hp, hh, jnp.abs(hp - hh), hp * hh], axis=1)
    x1 = jnp.dot(f, w1_ref[...], preferred_element_type=jnp.float32) + b1_ref[...]
    x2 = jnp.dot(x1, w2_ref[...], preferred_element_type=jnp.float32) + b2_ref[...]
    out_ref[...] = (jnp.dot(x2, w3_ref[...], preferred_element_type=jnp.float32)
                    + b3_ref[...])


def _mlp_tc(hp, hh, W1, b1, W2, b2, W3p, b3p):
    BM = 512
    return pl.pallas_call(
        _mlp_body,
        grid=(_B // BM,),
        in_specs=[
            pl.BlockSpec((BM, _D), lambda i: (i, 0)),
            pl.BlockSpec((BM, _D), lambda i: (i, 0)),
            pl.BlockSpec((4 * _D, _FC), lambda i: (0, 0)),
            pl.BlockSpec((1, _FC), lambda i: (0, 0)),
            pl.BlockSpec((_FC, _FC), lambda i: (0, 0)),
            pl.BlockSpec((1, _FC), lambda i: (0, 0)),
            pl.BlockSpec((_FC, _D), lambda i: (0, 0)),
            pl.BlockSpec((1, _D), lambda i: (0, 0)),
        ],
        out_specs=pl.BlockSpec((BM, _D), lambda i: (i, 0)),
        out_shape=jax.ShapeDtypeStruct((_B, _D), jnp.float32),
    )(hp, hh, W1, b1.reshape(1, _FC), W2, b2.reshape(1, _FC), W3p, b3p)


def kernel(hypothesis_tokens, hypothesis_len, premise_tokens, premise_len,
           emb_table, W1, b1, W2, b2, W3, b3):
    h_tok = hypothesis_tokens.astype(jnp.int32).reshape(_B * _L)
    p_tok = premise_tokens.astype(jnp.int32).reshape(_B * _L)
    h_len = hypothesis_len.astype(jnp.int32)
    p_len = premise_len.astype(jnp.int32)
    hh, hp = _pool_sc(h_tok, h_len, p_tok, p_len, emb_table)
    W3p = jnp.pad(W3, ((0, 0), (0, _D - W3.shape[1])))
    b3p = jnp.pad(b3, (0, _D - b3.shape[0])).reshape(1, _D)
    out = _mlp_tc(hp, hh, W1, b1, W2, b2, W3p, b3p)
    return out[:, :W3.shape[1]]
